# baseline (speedup 1.0000x reference)
import functools

import jax
import jax.numpy as jnp
from jax import lax
from jax.experimental import pallas as pl
from jax.experimental.pallas import tpu as pltpu
from jax.experimental.pallas import tpu_sc as plsc

D = 32
M = 8192
N = 8192
T_M = 1024
T_N = 2048
N_STEPS = N // T_M
N_CHUNKS = M // T_N
LOSS_SCALE = 1.25 / (N * D)

SC_NW = 32
SC_BPW = N // SC_NW
SC_IDX_CHUNK = 128


def _tc_body(x_ref, x2_ref, embt2_ref, idx_ref, loss_ref):
    i = pl.program_id(0)
    x = x_ref[...].reshape(T_M, D)
    x2 = x2_ref[...]
    xb = x.astype(jnp.bfloat16)
    carry_q = jnp.zeros((T_M, 1), jnp.float32)
    best_j = jnp.zeros((T_M, 1), dtype=jnp.int32)
    best_d = jnp.zeros((T_M, 1), jnp.float32)
    for w in range(N_CHUNKS):
        e_c = embt2_ref[:, pl.ds(w * T_N, T_N)]
        e2_c = jnp.sum((e_c * 0.5) ** 2, axis=0, keepdims=True)
        mm2 = lax.dot_general(xb, e_c, (((1,), (0,)), ((), ())),
                              preferred_element_type=jnp.float32)
        d = (x2 + e2_c) - mm2
        S_N = 128
        runv = d[:, 0:S_N]
        runs = jnp.zeros((T_M, S_N), jnp.int32)
        for s in range(1, T_N // S_N):
            ds = d[:, s * S_N:(s + 1) * S_N]
            upd = ds < runv
            runv = jnp.where(upd, ds, runv)
            runs = jnp.where(upd, s, runs)
        w_min = jnp.min(runv, axis=1, keepdims=True)
        lidx = lax.broadcasted_iota(jnp.int32, (T_M, S_N), 1)
        jj = runs * S_N + lidx
        w_j = jnp.min(jnp.where(runv == w_min, jj, T_N),
                      axis=1, keepdims=True) + w * T_N
        w_q = w_min.astype(jnp.bfloat16).astype(jnp.float32)
        if w == 0:
            carry_q, best_j, best_d = w_q, w_j, w_min
        else:
            wins = w_min < carry_q
            carry_q = jnp.where(wins, w_q, carry_q)
            best_j = jnp.where(wins, w_j, best_j)
            best_d = jnp.where(wins, w_min, best_d)
    idx_ref[...] = best_j
    s = jnp.sum(best_d, axis=(0, 1), keepdims=True)

    @pl.when(i == 0)
    def _():
        loss_ref[...] = jnp.zeros((1, 1), jnp.float32)

    loss_ref[...] += s

    @pl.when(i == N_STEPS - 1)
    def _():
        loss_ref[...] = loss_ref[...] * LOSS_SCALE


_distance_argmin = pl.pallas_call(
    _tc_body,
    grid=(N_STEPS,),
    in_specs=[
        pl.BlockSpec((1, T_M, D), lambda i: (i, 0, 0)),
        pl.BlockSpec((T_M, 1), lambda i: (i, 0)),
        pl.BlockSpec((D, M), lambda i: (0, 0)),
    ],
    out_specs=[
        pl.BlockSpec((T_M, 1), lambda i: (i, 0)),
        pl.BlockSpec((1, 1), lambda i: (0, 0)),
    ],
    out_shape=[
        jax.ShapeDtypeStruct((N, 1), jnp.int32),
        jax.ShapeDtypeStruct((1, 1), jnp.float32),
    ],
)


@functools.cache
def _make_sc_gather():
    @functools.partial(
        pl.kernel,
        mesh=plsc.VectorSubcoreMesh(core_axis_name="c", subcore_axis_name="s"),
        out_type=jax.ShapeDtypeStruct((N, D), jnp.float32),
        scratch_types=[
            pltpu.VMEM((SC_BPW // SC_IDX_CHUNK, SC_IDX_CHUNK), jnp.int32),
            pltpu.VMEM((SC_BPW, D), jnp.float32),
            pltpu.SemaphoreType.DMA,
        ],
        compiler_params=pltpu.CompilerParams(use_tc_tiling_on_sc=False),
    )
    def _sc_gather(emb_hbm, idx_hbm, out_hbm, idx_v, rows_v, sem):
        wid = lax.axis_index("s") * 2 + lax.axis_index("c")
        base = wid * SC_BPW
        n_sub = SC_BPW // SC_IDX_CHUNK
        for j in range(n_sub):
            pltpu.sync_copy(
                idx_hbm.at[pl.ds(base + j * SC_IDX_CHUNK, SC_IDX_CHUNK)],
                idx_v.at[j])
        copies = [
            pltpu.async_copy(
                emb_hbm.at[idx_v.at[j]],
                rows_v.at[pl.ds(j * SC_IDX_CHUNK, SC_IDX_CHUNK)],
                sem)
            for j in range(n_sub)
        ]
        for cp in copies:
            cp.wait()
        pltpu.sync_copy(rows_v, out_hbm.at[pl.ds(base, SC_BPW)])

    return _sc_gather


def kernel(inputs, embedding):
    input_shape = inputs.shape
    x2 = jnp.sum(inputs ** 2, axis=-1).reshape(-1, 1)
    idx2, loss = _distance_argmin(inputs, x2, 2.0 * embedding.T)
    quantized = _make_sc_gather()(embedding, idx2.reshape(-1)).reshape(input_shape)
    quantized_st = inputs + lax.stop_gradient(quantized - inputs)
    return (loss.reshape(()), quantized_st, idx2)


# speedup vs baseline: 1.0188x; 1.0188x over previous
import functools

import jax
import jax.numpy as jnp
from jax import lax
from jax.experimental import pallas as pl
from jax.experimental.pallas import tpu as pltpu
from jax.experimental.pallas import tpu_sc as plsc

D = 32
M = 8192
N = 8192
T_M = 1024
T_N = 2048
N_STEPS = N // T_M
N_CHUNKS = M // T_N
LOSS_SCALE = 1.25 / (N * D)

SC_NW = 32
SC_BPW = N // SC_NW
SC_IDX_CHUNK = 128


def _tc_body(x_ref, x2_ref, embt2_ref, idx_ref, loss_ref, e2_ref):
    @pl.when(pl.program_id(0) == 0)
    def _():
        for w in range(N_CHUNKS):
            e_c = embt2_ref[:, pl.ds(w * T_N, T_N)]
            e2_ref[:, pl.ds(w * T_N, T_N)] = jnp.sum(
                (e_c * 0.5) ** 2, axis=0, keepdims=True)
    i = pl.program_id(0)
    x = x_ref[...].reshape(T_M, D)
    x2 = x2_ref[...]
    xb = x.astype(jnp.bfloat16)
    carry_q = jnp.zeros((T_M, 1), jnp.float32)
    best_j = jnp.zeros((T_M, 1), dtype=jnp.int32)
    best_d = jnp.zeros((T_M, 1), jnp.float32)
    for w in range(N_CHUNKS):
        e_c = embt2_ref[:, pl.ds(w * T_N, T_N)]
        e2_c = e2_ref[:, pl.ds(w * T_N, T_N)]
        mm2 = lax.dot_general(xb, e_c, (((1,), (0,)), ((), ())),
                              preferred_element_type=jnp.float32)
        d = (x2 + e2_c) - mm2
        S_N = 128
        runv = d[:, 0:S_N]
        runs = jnp.zeros((T_M, S_N), jnp.int32)
        for s in range(1, T_N // S_N):
            ds = d[:, s * S_N:(s + 1) * S_N]
            upd = ds < runv
            runv = jnp.where(upd, ds, runv)
            runs = jnp.where(upd, s, runs)
        w_min = jnp.min(runv, axis=1, keepdims=True)
        lidx = lax.broadcasted_iota(jnp.int32, (T_M, S_N), 1)
        jj = runs * S_N + lidx
        w_j = jnp.min(jnp.where(runv == w_min, jj, T_N),
                      axis=1, keepdims=True) + w * T_N
        w_q = w_min.astype(jnp.bfloat16).astype(jnp.float32)
        if w == 0:
            carry_q, best_j, best_d = w_q, w_j, w_min
        else:
            wins = w_min < carry_q
            carry_q = jnp.where(wins, w_q, carry_q)
            best_j = jnp.where(wins, w_j, best_j)
            best_d = jnp.where(wins, w_min, best_d)
    idx_ref[...] = best_j
    s = jnp.sum(best_d, axis=(0, 1), keepdims=True)

    @pl.when(i == 0)
    def _():
        loss_ref[...] = jnp.zeros((1, 1), jnp.float32)

    loss_ref[...] += s

    @pl.when(i == N_STEPS - 1)
    def _():
        loss_ref[...] = loss_ref[...] * LOSS_SCALE


_distance_argmin = pl.pallas_call(
    _tc_body,
    grid=(N_STEPS,),
    in_specs=[
        pl.BlockSpec((1, T_M, D), lambda i: (i, 0, 0)),
        pl.BlockSpec((T_M, 1), lambda i: (i, 0)),
        pl.BlockSpec((D, M), lambda i: (0, 0)),
    ],
    out_specs=[
        pl.BlockSpec((T_M, 1), lambda i: (i, 0)),
        pl.BlockSpec((1, 1), lambda i: (0, 0)),
    ],
    out_shape=[
        jax.ShapeDtypeStruct((N, 1), jnp.int32),
        jax.ShapeDtypeStruct((1, 1), jnp.float32),
    ],
    scratch_shapes=[pltpu.VMEM((1, M), jnp.float32)],
)


@functools.cache
def _make_sc_gather():
    @functools.partial(
        pl.kernel,
        mesh=plsc.VectorSubcoreMesh(core_axis_name="c", subcore_axis_name="s"),
        out_type=jax.ShapeDtypeStruct((N, D), jnp.float32),
        scratch_types=[
            pltpu.VMEM((SC_BPW // SC_IDX_CHUNK, SC_IDX_CHUNK), jnp.int32),
            pltpu.VMEM((SC_BPW, D), jnp.float32),
            pltpu.SemaphoreType.DMA,
        ],
        compiler_params=pltpu.CompilerParams(use_tc_tiling_on_sc=False),
    )
    def _sc_gather(emb_hbm, idx_hbm, out_hbm, idx_v, rows_v, sem):
        wid = lax.axis_index("s") * 2 + lax.axis_index("c")
        base = wid * SC_BPW
        n_sub = SC_BPW // SC_IDX_CHUNK
        for j in range(n_sub):
            pltpu.sync_copy(
                idx_hbm.at[pl.ds(base + j * SC_IDX_CHUNK, SC_IDX_CHUNK)],
                idx_v.at[j])
        copies = [
            pltpu.async_copy(
                emb_hbm.at[idx_v.at[j]],
                rows_v.at[pl.ds(j * SC_IDX_CHUNK, SC_IDX_CHUNK)],
                sem)
            for j in range(n_sub)
        ]
        for cp in copies:
            cp.wait()
        pltpu.sync_copy(rows_v, out_hbm.at[pl.ds(base, SC_BPW)])

    return _sc_gather


def kernel(inputs, embedding):
    input_shape = inputs.shape
    x2 = jnp.sum(inputs ** 2, axis=-1).reshape(-1, 1)
    idx2, loss = _distance_argmin(inputs, x2, 2.0 * embedding.T)
    quantized = _make_sc_gather()(embedding, idx2.reshape(-1)).reshape(input_shape)
    quantized_st = inputs + lax.stop_gradient(quantized - inputs)
    return (loss.reshape(()), quantized_st, idx2)
